# Initial kernel scaffold; baseline (speedup 1.0000x reference)
import jax
import jax.numpy as jnp
from jax.experimental import pallas as pl


def kernel(h, edge_index, e, WA, WB, WC, WD, WE, gamma, beta):
    raise NotImplementedError("write your pallas kernel here")


# speedup vs baseline: 1.3473x; 1.3473x over previous
import functools

import jax
import jax.numpy as jnp
from jax import lax
from jax.experimental import pallas as pl
from jax.experimental.pallas import tpu as pltpu
from jax.experimental.pallas import tpu_sc as plsc

N_NODES = 10000
N_EDGES = 320000
HIDDEN = 128
EPS = 1e-5

NC = 2
NS = 16
L = 16
NW = NC * NS
EPW = N_EDGES // NW
CHUNK = 80
NCHUNK = EPW // CHUNK
N_PAD = 10240
RPT = N_PAD // NS


def _proj_body(h_ref, wsrc_ref, wb_ref, s_ref, t_ref):
    hb = h_ref[...]
    s_ref[...] = jnp.dot(hb, wsrc_ref[...], preferred_element_type=jnp.float32)
    t_ref[...] = jnp.dot(hb, wb_ref[...], preferred_element_type=jnp.float32)


def _node_proj(h, wsrc, wb):
    blk = 2000
    grid = N_NODES // blk
    return pl.pallas_call(
        _proj_body,
        grid=(grid,),
        in_specs=[
            pl.BlockSpec((blk, HIDDEN), lambda i: (i, 0)),
            pl.BlockSpec((HIDDEN, 2 * HIDDEN), lambda i: (0, 0)),
            pl.BlockSpec((HIDDEN, HIDDEN), lambda i: (0, 0)),
        ],
        out_specs=[
            pl.BlockSpec((blk, 2 * HIDDEN), lambda i: (i, 0)),
            pl.BlockSpec((blk, HIDDEN), lambda i: (i, 0)),
        ],
        out_shape=[
            jax.ShapeDtypeStruct((N_NODES, 2 * HIDDEN), jnp.float32),
            jax.ShapeDtypeStruct((N_NODES, HIDDEN), jnp.float32),
        ],
    )(h, wsrc, wb)


def _ee_body(e_ref, we_ref, out_ref):
    out_ref[...] = jnp.dot(e_ref[...], we_ref[...],
                           preferred_element_type=jnp.float32)


def _edge_proj(e, we):
    blk = 2000
    grid = N_EDGES // blk
    return pl.pallas_call(
        _ee_body,
        grid=(grid,),
        in_specs=[
            pl.BlockSpec((blk, HIDDEN), lambda i: (i, 0)),
            pl.BlockSpec((HIDDEN, HIDDEN), lambda i: (0, 0)),
        ],
        out_specs=pl.BlockSpec((blk, HIDDEN), lambda i: (i, 0)),
        out_shape=jax.ShapeDtypeStruct((N_EDGES, HIDDEN), jnp.float32),
    )(e, we)


def _sc_edge_body(s_hbm, t_hbm, ee_hbm, src_hbm, dst_hbm, out_hbm,
                  src_v, dst_v, s_v, t_v, m_v, acc, sem):
    c = lax.axis_index("c")
    s = lax.axis_index("s")
    wid = s * NC + c
    base = wid * EPW
    row0 = s * RPT

    zero = jnp.zeros((L,), jnp.float32)

    def zrow(i, carry):
        for j in range(HIDDEN // L):
            m_v[i, pl.ds(j * L, L)] = zero
        return carry

    lax.fori_loop(0, CHUNK, zrow, 0)
    for r in range(RPT // CHUNK):
        pltpu.sync_copy(m_v, acc.at[pl.ds(row0 + r * CHUNK, CHUNK)])
    plsc.subcore_barrier()

    def chunk_body(i, carry):
        off = base + i * CHUNK
        pltpu.sync_copy(src_hbm.at[pl.ds(off, CHUNK)], src_v)
        pltpu.sync_copy(dst_hbm.at[pl.ds(off, CHUNK)], dst_v)
        pltpu.sync_copy(ee_hbm.at[pl.ds(off, CHUNK)], m_v)
        pltpu.async_copy(s_hbm.at[src_v], s_v, sem).wait()
        pltpu.async_copy(t_hbm.at[dst_v], t_v, sem).wait()

        def edge_body(k, carry2):
            for j in range(HIDDEN // L):
                a = s_v[k, pl.ds(j * L, L)]
                cc = s_v[k, pl.ds(HIDDEN + j * L, L)]
                b = t_v[k, pl.ds(j * L, L)]
                ee = m_v[k, pl.ds(j * L, L)]
                x = a + b + ee
                gate = 1.0 / (1.0 + jnp.exp(-x))
                m_v[k, pl.ds(j * L, L)] = cc * gate
            return carry2

        lax.fori_loop(0, CHUNK, edge_body, 0)
        pltpu.sync_copy(m_v, acc.at[dst_v], add=True)
        return carry

    lax.fori_loop(0, NCHUNK, chunk_body, 0)
    plsc.subcore_barrier()

    pltpu.sync_copy(acc.at[pl.ds(row0, RPT)], out_hbm.at[c, pl.ds(row0, RPT)])


def _sc_edge(s_tab, t_tab, ee, src, dst):
    mesh = plsc.VectorSubcoreMesh(core_axis_name="c", subcore_axis_name="s")
    fn = functools.partial(
        pl.kernel,
        out_type=jax.ShapeDtypeStruct((NC, N_PAD, HIDDEN), jnp.float32),
        mesh=mesh,
        scratch_types=[
            pltpu.VMEM((CHUNK,), jnp.int32),
            pltpu.VMEM((CHUNK,), jnp.int32),
            pltpu.VMEM((CHUNK, 2 * HIDDEN), jnp.float32),
            pltpu.VMEM((CHUNK, HIDDEN), jnp.float32),
            pltpu.VMEM((CHUNK, HIDDEN), jnp.float32),
            pltpu.VMEM_SHARED((N_PAD, HIDDEN), jnp.float32),
            pltpu.SemaphoreType.DMA,
        ],
    )(_sc_edge_body)
    return fn(s_tab, t_tab, ee, src, dst)


def _final_body(h_ref, wd_ref, ms_ref, gamma_ref, beta_ref, out_ref):
    hn = jnp.dot(h_ref[...], wd_ref[...], preferred_element_type=jnp.float32)
    ms = ms_ref[...]
    hn = hn + ms[0, :N_NODES] + ms[1, :N_NODES]
    mean = jnp.mean(hn, axis=0, keepdims=True)
    xc = hn - mean
    var = jnp.mean(xc * xc, axis=0, keepdims=True)
    y = xc * lax.rsqrt(var + EPS) * gamma_ref[...] + beta_ref[...]
    out_ref[...] = jnp.maximum(y, 0.0)


def _final(h, wd, msum, gamma, beta):
    return pl.pallas_call(
        _final_body,
        out_shape=jax.ShapeDtypeStruct((N_NODES, HIDDEN), jnp.float32),
    )(h, wd, msum, gamma, beta)


def kernel(h, edge_index, e, WA, WB, WC, WD, WE, gamma, beta):
    src = edge_index[0].astype(jnp.int32)
    dst = edge_index[1].astype(jnp.int32)
    wsrc = jnp.concatenate([WA.T, WC.T], axis=1)
    s_tab, t_tab = _node_proj(h, wsrc, WB.T)
    ee = _edge_proj(e, WE.T)
    msum = _sc_edge(s_tab, t_tab, ee, src, dst)
    h_out = _final(h, WD.T, msum, gamma.reshape(1, HIDDEN),
                   beta.reshape(1, HIDDEN))
    return (h_out, ee)


# speedup vs baseline: 3.2906x; 2.4423x over previous
import functools

import jax
import jax.numpy as jnp
from jax import lax
from jax.experimental import pallas as pl
from jax.experimental.pallas import tpu as pltpu
from jax.experimental.pallas import tpu_sc as plsc

N_NODES = 10000
N_EDGES = 320000
HIDDEN = 128
EPS = 1e-5

NC = 2
NS = 16
L = 16
NW = NC * NS
EPW = N_EDGES // NW
CHUNK = 80
NCHUNK = EPW // CHUNK
N_PAD = 10240
RPT = N_PAD // NS


def _proj_body(h_ref, wsrc_ref, wb_ref, s_ref, t_ref):
    hb = h_ref[...]
    s_ref[...] = jnp.dot(hb, wsrc_ref[...], preferred_element_type=jnp.float32)
    t_ref[...] = jnp.dot(hb, wb_ref[...], preferred_element_type=jnp.float32)


def _node_proj(h, wsrc, wb):
    blk = 2000
    grid = N_NODES // blk
    return pl.pallas_call(
        _proj_body,
        grid=(grid,),
        in_specs=[
            pl.BlockSpec((blk, HIDDEN), lambda i: (i, 0)),
            pl.BlockSpec((HIDDEN, 2 * HIDDEN), lambda i: (0, 0)),
            pl.BlockSpec((HIDDEN, HIDDEN), lambda i: (0, 0)),
        ],
        out_specs=[
            pl.BlockSpec((blk, 2 * HIDDEN), lambda i: (i, 0)),
            pl.BlockSpec((blk, HIDDEN), lambda i: (i, 0)),
        ],
        out_shape=[
            jax.ShapeDtypeStruct((N_NODES, 2 * HIDDEN), jnp.float32),
            jax.ShapeDtypeStruct((N_NODES, HIDDEN), jnp.float32),
        ],
    )(h, wsrc, wb)


def _ee_body(e_ref, we_ref, out_ref):
    out_ref[...] = jnp.dot(e_ref[...], we_ref[...],
                           preferred_element_type=jnp.float32)


def _edge_proj(e, we):
    blk = 2000
    grid = N_EDGES // blk
    return pl.pallas_call(
        _ee_body,
        grid=(grid,),
        in_specs=[
            pl.BlockSpec((blk, HIDDEN), lambda i: (i, 0)),
            pl.BlockSpec((HIDDEN, HIDDEN), lambda i: (0, 0)),
        ],
        out_specs=pl.BlockSpec((blk, HIDDEN), lambda i: (i, 0)),
        out_shape=jax.ShapeDtypeStruct((N_EDGES, HIDDEN), jnp.float32),
    )(e, we)


def _sc_edge_body(s_hbm, t_hbm, ee_hbm, src_hbm, dst_hbm, out_hbm,
                  src_v, dst_v, s_v, t_v, m_v, acc, sem):
    c = lax.axis_index("c")
    s = lax.axis_index("s")
    wid = s * NC + c
    base = wid * EPW
    row0 = s * RPT

    zero = jnp.zeros((L,), jnp.float32)

    def zrow(i, carry):
        for j in range(HIDDEN // L):
            m_v[i, pl.ds(j * L, L)] = zero
        return carry

    lax.fori_loop(0, CHUNK, zrow, 0)
    for r in range(RPT // CHUNK):
        pltpu.sync_copy(m_v, acc.at[pl.ds(row0 + r * CHUNK, CHUNK)])
    plsc.subcore_barrier()

    def chunk_body(i, carry):
        off = base + i * CHUNK
        pltpu.sync_copy(src_hbm.at[pl.ds(off, CHUNK)], src_v)
        pltpu.sync_copy(dst_hbm.at[pl.ds(off, CHUNK)], dst_v)
        pltpu.sync_copy(ee_hbm.at[pl.ds(off, CHUNK)], m_v)
        pltpu.async_copy(s_hbm.at[src_v], s_v, sem).wait()
        pltpu.async_copy(t_hbm.at[dst_v], t_v, sem).wait()

        @plsc.parallel_loop(0, CHUNK, step=1, unroll=4)
        def edge_body(k):
            for j in range(HIDDEN // L):
                a = s_v[k, pl.ds(j * L, L)]
                cc = s_v[k, pl.ds(HIDDEN + j * L, L)]
                b = t_v[k, pl.ds(j * L, L)]
                ee = m_v[k, pl.ds(j * L, L)]
                x = a + b + ee
                gate = 1.0 / (1.0 + jnp.exp(-x))
                m_v[k, pl.ds(j * L, L)] = cc * gate
        pltpu.sync_copy(m_v, acc.at[dst_v], add=True)
        return carry

    lax.fori_loop(0, NCHUNK, chunk_body, 0)
    plsc.subcore_barrier()

    pltpu.sync_copy(acc.at[pl.ds(row0, RPT)], out_hbm.at[c, pl.ds(row0, RPT)])


def _sc_edge(s_tab, t_tab, ee, src, dst):
    mesh = plsc.VectorSubcoreMesh(core_axis_name="c", subcore_axis_name="s")
    fn = functools.partial(
        pl.kernel,
        out_type=jax.ShapeDtypeStruct((NC, N_PAD, HIDDEN), jnp.float32),
        mesh=mesh,
        scratch_types=[
            pltpu.VMEM((CHUNK,), jnp.int32),
            pltpu.VMEM((CHUNK,), jnp.int32),
            pltpu.VMEM((CHUNK, 2 * HIDDEN), jnp.float32),
            pltpu.VMEM((CHUNK, HIDDEN), jnp.float32),
            pltpu.VMEM((CHUNK, HIDDEN), jnp.float32),
            pltpu.VMEM_SHARED((N_PAD, HIDDEN), jnp.float32),
            pltpu.SemaphoreType.DMA,
        ],
    )(_sc_edge_body)
    return fn(s_tab, t_tab, ee, src, dst)


def _final_body(h_ref, wd_ref, ms_ref, gamma_ref, beta_ref, out_ref):
    hn = jnp.dot(h_ref[...], wd_ref[...], preferred_element_type=jnp.float32)
    ms = ms_ref[...]
    hn = hn + ms[0, :N_NODES] + ms[1, :N_NODES]
    mean = jnp.mean(hn, axis=0, keepdims=True)
    xc = hn - mean
    var = jnp.mean(xc * xc, axis=0, keepdims=True)
    y = xc * lax.rsqrt(var + EPS) * gamma_ref[...] + beta_ref[...]
    out_ref[...] = jnp.maximum(y, 0.0)


def _final(h, wd, msum, gamma, beta):
    return pl.pallas_call(
        _final_body,
        out_shape=jax.ShapeDtypeStruct((N_NODES, HIDDEN), jnp.float32),
    )(h, wd, msum, gamma, beta)


def kernel(h, edge_index, e, WA, WB, WC, WD, WE, gamma, beta):
    src = edge_index[0].astype(jnp.int32)
    dst = edge_index[1].astype(jnp.int32)
    wsrc = jnp.concatenate([WA.T, WC.T], axis=1)
    s_tab, t_tab = _node_proj(h, wsrc, WB.T)
    ee = _edge_proj(e, WE.T)
    msum = _sc_edge(s_tab, t_tab, ee, src, dst)
    h_out = _final(h, WD.T, msum, gamma.reshape(1, HIDDEN),
                   beta.reshape(1, HIDDEN))
    return (h_out, ee)


# speedup vs baseline: 3.3005x; 1.0030x over previous
import functools

import jax
import jax.numpy as jnp
from jax import lax
from jax.experimental import pallas as pl
from jax.experimental.pallas import tpu as pltpu
from jax.experimental.pallas import tpu_sc as plsc

N_NODES = 10000
N_EDGES = 320000
HIDDEN = 128
EPS = 1e-5

NC = 2
NS = 16
L = 16
NW = NC * NS
EPW = N_EDGES // NW
CHUNK = 80
NCHUNK = EPW // CHUNK
N_PAD = 10240
RPT = N_PAD // NS


def _proj_body(h_ref, wsrc_ref, wb_ref, s_ref, t_ref):
    hb = h_ref[...]
    s_ref[...] = jnp.dot(hb, wsrc_ref[...], preferred_element_type=jnp.float32)
    t_ref[...] = jnp.dot(hb, wb_ref[...], preferred_element_type=jnp.float32)


def _node_proj(h, wsrc, wb):
    blk = 2000
    grid = N_NODES // blk
    return pl.pallas_call(
        _proj_body,
        grid=(grid,),
        in_specs=[
            pl.BlockSpec((blk, HIDDEN), lambda i: (i, 0)),
            pl.BlockSpec((HIDDEN, 2 * HIDDEN), lambda i: (0, 0)),
            pl.BlockSpec((HIDDEN, HIDDEN), lambda i: (0, 0)),
        ],
        out_specs=[
            pl.BlockSpec((blk, 2 * HIDDEN), lambda i: (i, 0)),
            pl.BlockSpec((blk, HIDDEN), lambda i: (i, 0)),
        ],
        out_shape=[
            jax.ShapeDtypeStruct((N_NODES, 2 * HIDDEN), jnp.float32),
            jax.ShapeDtypeStruct((N_NODES, HIDDEN), jnp.float32),
        ],
    )(h, wsrc, wb)


def _ee_body(e_ref, we_ref, out_ref):
    out_ref[...] = jnp.dot(e_ref[...], we_ref[...],
                           preferred_element_type=jnp.float32)


def _edge_proj(e, we):
    blk = 2000
    grid = N_EDGES // blk
    return pl.pallas_call(
        _ee_body,
        grid=(grid,),
        in_specs=[
            pl.BlockSpec((blk, HIDDEN), lambda i: (i, 0)),
            pl.BlockSpec((HIDDEN, HIDDEN), lambda i: (0, 0)),
        ],
        out_specs=pl.BlockSpec((blk, HIDDEN), lambda i: (i, 0)),
        out_shape=jax.ShapeDtypeStruct((N_EDGES, HIDDEN), jnp.float32),
    )(e, we)


def _sc_edge_body(s_hbm, t_hbm, ee_hbm, src_hbm, dst_hbm, out_hbm,
                  src_v, dst_v, s_v, t_v, m_v, acc, sem):
    c = lax.axis_index("c")
    s = lax.axis_index("s")
    wid = s * NC + c
    base = wid * EPW
    row0 = s * RPT

    zero = jnp.zeros((L,), jnp.float32)

    def zrow(i, carry):
        for j in range(HIDDEN // L):
            m_v[i, pl.ds(j * L, L)] = zero
        return carry

    lax.fori_loop(0, CHUNK, zrow, 0)
    for r in range(RPT // CHUNK):
        pltpu.sync_copy(m_v, acc.at[pl.ds(row0 + r * CHUNK, CHUNK)])
    plsc.subcore_barrier()

    def chunk_body(i, carry):
        off = base + i * CHUNK
        pltpu.sync_copy(src_hbm.at[pl.ds(off, CHUNK)], src_v)
        pltpu.sync_copy(dst_hbm.at[pl.ds(off, CHUNK)], dst_v)
        pltpu.sync_copy(ee_hbm.at[pl.ds(off, CHUNK)], m_v)
        pltpu.async_copy(s_hbm.at[src_v], s_v, sem).wait()
        pltpu.async_copy(t_hbm.at[dst_v], t_v, sem).wait()

        @plsc.parallel_loop(0, CHUNK, step=1, unroll=8)
        def edge_body(k):
            for j in range(HIDDEN // L):
                a = s_v[k, pl.ds(j * L, L)]
                cc = s_v[k, pl.ds(HIDDEN + j * L, L)]
                b = t_v[k, pl.ds(j * L, L)]
                ee = m_v[k, pl.ds(j * L, L)]
                x = a + b + ee
                gate = 1.0 / (1.0 + jnp.exp(-x))
                m_v[k, pl.ds(j * L, L)] = cc * gate
        pltpu.sync_copy(m_v, acc.at[dst_v], add=True)
        return carry

    lax.fori_loop(0, NCHUNK, chunk_body, 0)
    plsc.subcore_barrier()

    pltpu.sync_copy(acc.at[pl.ds(row0, RPT)], out_hbm.at[c, pl.ds(row0, RPT)])


def _sc_edge(s_tab, t_tab, ee, src, dst):
    mesh = plsc.VectorSubcoreMesh(core_axis_name="c", subcore_axis_name="s")
    fn = functools.partial(
        pl.kernel,
        out_type=jax.ShapeDtypeStruct((NC, N_PAD, HIDDEN), jnp.float32),
        mesh=mesh,
        scratch_types=[
            pltpu.VMEM((CHUNK,), jnp.int32),
            pltpu.VMEM((CHUNK,), jnp.int32),
            pltpu.VMEM((CHUNK, 2 * HIDDEN), jnp.float32),
            pltpu.VMEM((CHUNK, HIDDEN), jnp.float32),
            pltpu.VMEM((CHUNK, HIDDEN), jnp.float32),
            pltpu.VMEM_SHARED((N_PAD, HIDDEN), jnp.float32),
            pltpu.SemaphoreType.DMA,
        ],
    )(_sc_edge_body)
    return fn(s_tab, t_tab, ee, src, dst)


def _final_body(h_ref, wd_ref, ms_ref, gamma_ref, beta_ref, out_ref):
    hn = jnp.dot(h_ref[...], wd_ref[...], preferred_element_type=jnp.float32)
    ms = ms_ref[...]
    hn = hn + ms[0, :N_NODES] + ms[1, :N_NODES]
    mean = jnp.mean(hn, axis=0, keepdims=True)
    xc = hn - mean
    var = jnp.mean(xc * xc, axis=0, keepdims=True)
    y = xc * lax.rsqrt(var + EPS) * gamma_ref[...] + beta_ref[...]
    out_ref[...] = jnp.maximum(y, 0.0)


def _final(h, wd, msum, gamma, beta):
    return pl.pallas_call(
        _final_body,
        out_shape=jax.ShapeDtypeStruct((N_NODES, HIDDEN), jnp.float32),
    )(h, wd, msum, gamma, beta)


def kernel(h, edge_index, e, WA, WB, WC, WD, WE, gamma, beta):
    src = edge_index[0].astype(jnp.int32)
    dst = edge_index[1].astype(jnp.int32)
    wsrc = jnp.concatenate([WA.T, WC.T], axis=1)
    s_tab, t_tab = _node_proj(h, wsrc, WB.T)
    ee = _edge_proj(e, WE.T)
    msum = _sc_edge(s_tab, t_tab, ee, src, dst)
    h_out = _final(h, WD.T, msum, gamma.reshape(1, HIDDEN),
                   beta.reshape(1, HIDDEN))
    return (h_out, ee)


# speedup vs baseline: 3.9188x; 1.1873x over previous
import functools

import jax
import jax.numpy as jnp
from jax import lax
from jax.experimental import pallas as pl
from jax.experimental.pallas import tpu as pltpu
from jax.experimental.pallas import tpu_sc as plsc

N_NODES = 10000
N_EDGES = 320000
HIDDEN = 128
EPS = 1e-5

NC = 2
NS = 16
L = 16
NW = NC * NS
EPW = N_EDGES // NW
CHUNK = 40
NCHUNK = EPW // CHUNK
N_PAD = 10240
RPT = N_PAD // NS


def _proj_body(h_ref, wsrc_ref, wb_ref, s_ref, t_ref):
    hb = h_ref[...]
    s_ref[...] = jnp.dot(hb, wsrc_ref[...], preferred_element_type=jnp.float32)
    t_ref[...] = jnp.dot(hb, wb_ref[...], preferred_element_type=jnp.float32)


def _node_proj(h, wsrc, wb):
    blk = 2000
    grid = N_NODES // blk
    return pl.pallas_call(
        _proj_body,
        grid=(grid,),
        in_specs=[
            pl.BlockSpec((blk, HIDDEN), lambda i: (i, 0)),
            pl.BlockSpec((HIDDEN, 2 * HIDDEN), lambda i: (0, 0)),
            pl.BlockSpec((HIDDEN, HIDDEN), lambda i: (0, 0)),
        ],
        out_specs=[
            pl.BlockSpec((blk, 2 * HIDDEN), lambda i: (i, 0)),
            pl.BlockSpec((blk, HIDDEN), lambda i: (i, 0)),
        ],
        out_shape=[
            jax.ShapeDtypeStruct((N_NODES, 2 * HIDDEN), jnp.float32),
            jax.ShapeDtypeStruct((N_NODES, HIDDEN), jnp.float32),
        ],
    )(h, wsrc, wb)


def _ee_body(e_ref, we_ref, out_ref):
    out_ref[...] = jnp.dot(e_ref[...], we_ref[...],
                           preferred_element_type=jnp.float32)


def _edge_proj(e, we):
    blk = 2000
    grid = N_EDGES // blk
    return pl.pallas_call(
        _ee_body,
        grid=(grid,),
        in_specs=[
            pl.BlockSpec((blk, HIDDEN), lambda i: (i, 0)),
            pl.BlockSpec((HIDDEN, HIDDEN), lambda i: (0, 0)),
        ],
        out_specs=pl.BlockSpec((blk, HIDDEN), lambda i: (i, 0)),
        out_shape=jax.ShapeDtypeStruct((N_EDGES, HIDDEN), jnp.float32),
    )(e, we)


def _sc_edge_body(s_hbm, t_hbm, ee_hbm, src_hbm, dst_hbm, out_hbm,
                  src0, dst0, src1, dst1, s0, s1, t0, t1, m0, m1, acc,
                  semi0, semi1, seme0, seme1, semb0, semb1, sems0, sems1):
    c = lax.axis_index("c")
    s = lax.axis_index("s")
    wid = s * NC + c
    base = wid * EPW
    row0 = s * RPT

    srcv, dstv = [src0, src1], [dst0, dst1]
    sv, tv, mv = [s0, s1], [t0, t1], [m0, m1]
    semi, seme = [semi0, semi1], [seme0, seme1]
    semb, sems = [semb0, semb1], [sems0, sems1]

    zero = jnp.zeros((L,), jnp.float32)

    def zrow(i, carry):
        for j in range(HIDDEN // L):
            m0[i, pl.ds(j * L, L)] = zero
        return carry

    lax.fori_loop(0, CHUNK, zrow, 0)
    for r in range(RPT // CHUNK):
        pltpu.sync_copy(m0, acc.at[pl.ds(row0 + r * CHUNK, CHUNK)])
    plsc.subcore_barrier()

    def issue_a(ci, p):
        off = base + ci * CHUNK
        pltpu.async_copy(src_hbm.at[pl.ds(off, CHUNK)], srcv[p], semi[p])
        pltpu.async_copy(dst_hbm.at[pl.ds(off, CHUNK)], dstv[p], semi[p])
        pltpu.async_copy(ee_hbm.at[pl.ds(off, CHUNK)], mv[p], seme[p])

    def wait_a_idx(p):
        pltpu.make_async_copy(src_hbm.at[pl.ds(0, CHUNK)], srcv[p], semi[p]).wait()
        pltpu.make_async_copy(dst_hbm.at[pl.ds(0, CHUNK)], dstv[p], semi[p]).wait()

    def issue_b(p):
        pltpu.async_copy(s_hbm.at[srcv[p]], sv[p], semb[p])
        pltpu.async_copy(t_hbm.at[dstv[p]], tv[p], semb[p])

    def wait_b(p):
        pltpu.make_async_copy(s_hbm.at[pl.ds(0, CHUNK)], sv[p], semb[p]).wait()
        pltpu.make_async_copy(t_hbm.at[pl.ds(0, CHUNK)], tv[p], semb[p]).wait()

    def wait_ee(p):
        pltpu.make_async_copy(ee_hbm.at[pl.ds(0, CHUNK)], mv[p], seme[p]).wait()

    def wait_scat(p):
        pltpu.make_async_copy(ee_hbm.at[pl.ds(0, CHUNK)], mv[p], sems[p]).wait()

    def half(ci, p):
        wait_b(p)
        wait_ee(p)

        @plsc.parallel_loop(0, CHUNK, step=1, unroll=4)
        def edge_body(k):
            for j in range(HIDDEN // L):
                a = sv[p][k, pl.ds(j * L, L)]
                cc = sv[p][k, pl.ds(HIDDEN + j * L, L)]
                b = tv[p][k, pl.ds(j * L, L)]
                ee = mv[p][k, pl.ds(j * L, L)]
                x = a + b + ee
                gate = 1.0 / (1.0 + jnp.exp(-x))
                mv[p][k, pl.ds(j * L, L)] = cc * gate

        pltpu.async_copy(mv[p], acc.at[dstv[p]], sems[p], add=True)

        @pl.when(ci + 1 < NCHUNK)
        def _():
            wait_a_idx(1 - p)
            issue_b(1 - p)

        wait_scat(p)

        @pl.when(ci + 2 < NCHUNK)
        def _():
            issue_a(ci + 2, p)

    issue_a(0, 0)
    issue_a(1, 1)
    wait_a_idx(0)
    issue_b(0)

    def pair_body(it, carry):
        half(2 * it, 0)
        half(2 * it + 1, 1)
        return carry

    lax.fori_loop(0, NCHUNK // 2, pair_body, 0)
    plsc.subcore_barrier()

    pltpu.sync_copy(acc.at[pl.ds(row0, RPT)], out_hbm.at[c, pl.ds(row0, RPT)])


def _sc_edge(s_tab, t_tab, ee, src, dst):
    mesh = plsc.VectorSubcoreMesh(core_axis_name="c", subcore_axis_name="s")
    fn = functools.partial(
        pl.kernel,
        out_type=jax.ShapeDtypeStruct((NC, N_PAD, HIDDEN), jnp.float32),
        mesh=mesh,
        scratch_types=[
            pltpu.VMEM((CHUNK,), jnp.int32),
            pltpu.VMEM((CHUNK,), jnp.int32),
            pltpu.VMEM((CHUNK,), jnp.int32),
            pltpu.VMEM((CHUNK,), jnp.int32),
            pltpu.VMEM((CHUNK, 2 * HIDDEN), jnp.float32),
            pltpu.VMEM((CHUNK, 2 * HIDDEN), jnp.float32),
            pltpu.VMEM((CHUNK, HIDDEN), jnp.float32),
            pltpu.VMEM((CHUNK, HIDDEN), jnp.float32),
            pltpu.VMEM((CHUNK, HIDDEN), jnp.float32),
            pltpu.VMEM((CHUNK, HIDDEN), jnp.float32),
            pltpu.VMEM_SHARED((N_PAD, HIDDEN), jnp.float32),
            pltpu.SemaphoreType.DMA,
            pltpu.SemaphoreType.DMA,
            pltpu.SemaphoreType.DMA,
            pltpu.SemaphoreType.DMA,
            pltpu.SemaphoreType.DMA,
            pltpu.SemaphoreType.DMA,
            pltpu.SemaphoreType.DMA,
            pltpu.SemaphoreType.DMA,
        ],
    )(_sc_edge_body)
    return fn(s_tab, t_tab, ee, src, dst)


def _final_body(h_ref, wd_ref, ms_ref, gamma_ref, beta_ref, out_ref):
    hn = jnp.dot(h_ref[...], wd_ref[...], preferred_element_type=jnp.float32)
    ms = ms_ref[...]
    hn = hn + ms[0, :N_NODES] + ms[1, :N_NODES]
    mean = jnp.mean(hn, axis=0, keepdims=True)
    xc = hn - mean
    var = jnp.mean(xc * xc, axis=0, keepdims=True)
    y = xc * lax.rsqrt(var + EPS) * gamma_ref[...] + beta_ref[...]
    out_ref[...] = jnp.maximum(y, 0.0)


def _final(h, wd, msum, gamma, beta):
    return pl.pallas_call(
        _final_body,
        out_shape=jax.ShapeDtypeStruct((N_NODES, HIDDEN), jnp.float32),
    )(h, wd, msum, gamma, beta)


def kernel(h, edge_index, e, WA, WB, WC, WD, WE, gamma, beta):
    src = edge_index[0].astype(jnp.int32)
    dst = edge_index[1].astype(jnp.int32)
    wsrc = jnp.concatenate([WA.T, WC.T], axis=1)
    s_tab, t_tab = _node_proj(h, wsrc, WB.T)
    ee = _edge_proj(e, WE.T)
    msum = _sc_edge(s_tab, t_tab, ee, src, dst)
    h_out = _final(h, WD.T, msum, gamma.reshape(1, HIDDEN),
                   beta.reshape(1, HIDDEN))
    return (h_out, ee)


# speedup vs baseline: 4.5740x; 1.1672x over previous
import functools

import jax
import jax.numpy as jnp
from jax import lax
from jax.experimental import pallas as pl
from jax.experimental.pallas import tpu as pltpu
from jax.experimental.pallas import tpu_sc as plsc

N_NODES = 10000
N_EDGES = 320000
HIDDEN = 128
EPS = 1e-5

NC = 2
NS = 16
L = 16
NW = NC * NS
EPW = N_EDGES // NW
CHUNK = 40
NCHUNK = EPW // CHUNK
N_PAD = 10240
RPT = N_PAD // NS


def _proj_body(h_ref, wsrc_ref, wb_ref, s_ref, t_ref):
    hb = h_ref[...]
    s_ref[...] = jnp.dot(hb, wsrc_ref[...], preferred_element_type=jnp.float32)
    t_ref[...] = jnp.dot(hb, wb_ref[...], preferred_element_type=jnp.float32)


def _node_proj(h, wsrc, wb):
    blk = 2000
    grid = N_NODES // blk
    return pl.pallas_call(
        _proj_body,
        grid=(grid,),
        in_specs=[
            pl.BlockSpec((blk, HIDDEN), lambda i: (i, 0)),
            pl.BlockSpec((HIDDEN, 2 * HIDDEN), lambda i: (0, 0)),
            pl.BlockSpec((HIDDEN, HIDDEN), lambda i: (0, 0)),
        ],
        out_specs=[
            pl.BlockSpec((blk, 2 * HIDDEN), lambda i: (i, 0)),
            pl.BlockSpec((blk, HIDDEN), lambda i: (i, 0)),
        ],
        out_shape=[
            jax.ShapeDtypeStruct((N_NODES, 2 * HIDDEN), jnp.float32),
            jax.ShapeDtypeStruct((N_NODES, HIDDEN), jnp.float32),
        ],
    )(h, wsrc, wb)


def _ee_body(e_ref, we_ref, out_ref):
    out_ref[...] = jnp.dot(e_ref[...], we_ref[...],
                           preferred_element_type=jnp.float32)


def _edge_proj(e, we):
    blk = 2000
    grid = N_EDGES // blk
    return pl.pallas_call(
        _ee_body,
        grid=(grid,),
        in_specs=[
            pl.BlockSpec((blk, HIDDEN), lambda i: (i, 0)),
            pl.BlockSpec((HIDDEN, HIDDEN), lambda i: (0, 0)),
        ],
        out_specs=pl.BlockSpec((blk, HIDDEN), lambda i: (i, 0)),
        out_shape=jax.ShapeDtypeStruct((N_EDGES, HIDDEN), jnp.float32),
    )(e, we)


def _sc_edge_body(s_hbm, t_hbm, ee_hbm, src_hbm, dst_hbm, out_hbm,
                  src0, dst0, src1, dst1, s0, s1, t0, t1, m0, m1, acc,
                  semi0, semi1, seme0, seme1, semb0, semb1, sems0, sems1):
    c = lax.axis_index("c")
    s = lax.axis_index("s")
    wid = s * NC + c
    base = wid * EPW
    row0 = s * RPT

    srcv, dstv = [src0, src1], [dst0, dst1]
    sv, tv, mv = [s0, s1], [t0, t1], [m0, m1]
    semi, seme = [semi0, semi1], [seme0, seme1]
    semb, sems = [semb0, semb1], [sems0, sems1]

    zero = jnp.zeros((L,), jnp.float32)

    def zrow(i, carry):
        for j in range(HIDDEN // L):
            m0[i, pl.ds(j * L, L)] = zero
        return carry

    lax.fori_loop(0, CHUNK, zrow, 0)
    for r in range(RPT // CHUNK):
        pltpu.sync_copy(m0, acc.at[pl.ds(row0 + r * CHUNK, CHUNK)])
    plsc.subcore_barrier()

    def issue_a(ci, p):
        off = base + ci * CHUNK
        pltpu.async_copy(src_hbm.at[pl.ds(off, CHUNK)], srcv[p], semi[p])
        pltpu.async_copy(dst_hbm.at[pl.ds(off, CHUNK)], dstv[p], semi[p])
        pltpu.async_copy(ee_hbm.at[pl.ds(off, CHUNK)], mv[p], seme[p])

    def wait_a_idx(p):
        pltpu.make_async_copy(src_hbm.at[pl.ds(0, CHUNK)], srcv[p], semi[p]).wait()
        pltpu.make_async_copy(dst_hbm.at[pl.ds(0, CHUNK)], dstv[p], semi[p]).wait()

    def issue_b(p):
        pltpu.async_copy(s_hbm.at[srcv[p]], sv[p], semb[p])
        pltpu.async_copy(t_hbm.at[dstv[p]], tv[p], semb[p])

    def wait_b(p):
        pltpu.make_async_copy(s_hbm.at[pl.ds(0, CHUNK)], sv[p], semb[p]).wait()
        pltpu.make_async_copy(t_hbm.at[pl.ds(0, CHUNK)], tv[p], semb[p]).wait()

    def wait_ee(p):
        pltpu.make_async_copy(ee_hbm.at[pl.ds(0, CHUNK)], mv[p], seme[p]).wait()

    def wait_scat(p):
        pltpu.make_async_copy(ee_hbm.at[pl.ds(0, CHUNK)], mv[p], sems[p]).wait()

    def half(ci, p):
        wait_b(p)

        @pl.when(ci + 1 < NCHUNK)
        def _():
            wait_a_idx(1 - p)
            issue_b(1 - p)

        wait_ee(p)

        @plsc.parallel_loop(0, CHUNK, step=1, unroll=4)
        def edge_body(k):
            for j in range(HIDDEN // L):
                a = sv[p][k, pl.ds(j * L, L)]
                cc = sv[p][k, pl.ds(HIDDEN + j * L, L)]
                b = tv[p][k, pl.ds(j * L, L)]
                ee = mv[p][k, pl.ds(j * L, L)]
                x = a + b + ee
                gate = 1.0 / (1.0 + jnp.exp(-x))
                mv[p][k, pl.ds(j * L, L)] = cc * gate

        pltpu.async_copy(mv[p], acc.at[dstv[p]], sems[p], add=True)
        wait_scat(p)

        @pl.when(ci + 2 < NCHUNK)
        def _():
            issue_a(ci + 2, p)

    issue_a(0, 0)
    issue_a(1, 1)
    wait_a_idx(0)
    issue_b(0)

    def pair_body(it, carry):
        half(2 * it, 0)
        half(2 * it + 1, 1)
        return carry

    lax.fori_loop(0, NCHUNK // 2, pair_body, 0)
    plsc.subcore_barrier()

    pltpu.sync_copy(acc.at[pl.ds(row0, RPT)], out_hbm.at[c, pl.ds(row0, RPT)])


def _sc_edge(s_tab, t_tab, ee, src, dst):
    mesh = plsc.VectorSubcoreMesh(core_axis_name="c", subcore_axis_name="s")
    fn = functools.partial(
        pl.kernel,
        out_type=jax.ShapeDtypeStruct((NC, N_PAD, HIDDEN), jnp.float32),
        mesh=mesh,
        scratch_types=[
            pltpu.VMEM((CHUNK,), jnp.int32),
            pltpu.VMEM((CHUNK,), jnp.int32),
            pltpu.VMEM((CHUNK,), jnp.int32),
            pltpu.VMEM((CHUNK,), jnp.int32),
            pltpu.VMEM((CHUNK, 2 * HIDDEN), jnp.float32),
            pltpu.VMEM((CHUNK, 2 * HIDDEN), jnp.float32),
            pltpu.VMEM((CHUNK, HIDDEN), jnp.float32),
            pltpu.VMEM((CHUNK, HIDDEN), jnp.float32),
            pltpu.VMEM((CHUNK, HIDDEN), jnp.float32),
            pltpu.VMEM((CHUNK, HIDDEN), jnp.float32),
            pltpu.VMEM_SHARED((N_PAD, HIDDEN), jnp.float32),
            pltpu.SemaphoreType.DMA,
            pltpu.SemaphoreType.DMA,
            pltpu.SemaphoreType.DMA,
            pltpu.SemaphoreType.DMA,
            pltpu.SemaphoreType.DMA,
            pltpu.SemaphoreType.DMA,
            pltpu.SemaphoreType.DMA,
            pltpu.SemaphoreType.DMA,
        ],
    )(_sc_edge_body)
    return fn(s_tab, t_tab, ee, src, dst)


def _final_body(h_ref, wd_ref, ms_ref, gamma_ref, beta_ref, out_ref):
    hn = jnp.dot(h_ref[...], wd_ref[...], preferred_element_type=jnp.float32)
    ms = ms_ref[...]
    hn = hn + ms[0, :N_NODES] + ms[1, :N_NODES]
    mean = jnp.mean(hn, axis=0, keepdims=True)
    xc = hn - mean
    var = jnp.mean(xc * xc, axis=0, keepdims=True)
    y = xc * lax.rsqrt(var + EPS) * gamma_ref[...] + beta_ref[...]
    out_ref[...] = jnp.maximum(y, 0.0)


def _final(h, wd, msum, gamma, beta):
    return pl.pallas_call(
        _final_body,
        out_shape=jax.ShapeDtypeStruct((N_NODES, HIDDEN), jnp.float32),
    )(h, wd, msum, gamma, beta)


def kernel(h, edge_index, e, WA, WB, WC, WD, WE, gamma, beta):
    src = edge_index[0].astype(jnp.int32)
    dst = edge_index[1].astype(jnp.int32)
    wsrc = jnp.concatenate([WA.T, WC.T], axis=1)
    s_tab, t_tab = _node_proj(h, wsrc, WB.T)
    ee = _edge_proj(e, WE.T)
    msum = _sc_edge(s_tab, t_tab, ee, src, dst)
    h_out = _final(h, WD.T, msum, gamma.reshape(1, HIDDEN),
                   beta.reshape(1, HIDDEN))
    return (h_out, ee)
